# baseline (device time: 61198 ns/iter reference)
import jax
import jax.numpy as jnp
from jax import lax
from jax.experimental import pallas as pl
from jax.experimental.pallas import tpu as pltpu

N_DEV = 4
CHUNK = 256


def kernel(x):
    m, n = x.shape
    nc = m // CHUNK

    def body(x_hbm, out_hbm, xbuf, res, offset_ref, send_buf, recv_buf,
             in_sems, out_sems, send_sem, recv_sem):
        my = lax.axis_index("i")

        def in_copy(i):
            return pltpu.make_async_copy(
                x_hbm.at[pl.ds(i * CHUNK, CHUNK), :],
                xbuf.at[i % 2],
                in_sems.at[i % 2],
            )

        def out_copy(i):
            return pltpu.make_async_copy(
                res.at[pl.ds(i * CHUNK, CHUNK), :],
                out_hbm.at[pl.ds(i * CHUNK, CHUNK), :],
                out_sems.at[i % 2],
            )

        row = lax.broadcasted_iota(jnp.int32, (CHUNK, CHUNK), 0)
        col = lax.broadcasted_iota(jnp.int32, (CHUNK, CHUNK), 1)
        tri = (row >= col).astype(jnp.bfloat16)

        in_copy(0).start()
        carry = jnp.zeros((1, n), jnp.float32)
        for i in range(nc):
            if i + 1 < nc:
                in_copy(i + 1).start()
            in_copy(i).wait()
            chunk = xbuf[i % 2].astype(jnp.bfloat16)
            cum = lax.dot_general(
                tri, chunk,
                (((1,), (0,)), ((), ())),
                preferred_element_type=jnp.float32,
            )
            cum = cum + carry
            res[pl.ds(i * CHUNK, CHUNK), :] = cum.astype(jnp.bfloat16)
            carry = cum[CHUNK - 1:CHUNK, :]

        total = carry

        @pl.when(my == 0)
        def _():
            offset_ref[...] = jnp.zeros((1, n), jnp.float32)

        @pl.when(my > 0)
        def _():
            recv = pltpu.make_async_remote_copy(
                src_ref=send_buf,
                dst_ref=recv_buf,
                send_sem=send_sem,
                recv_sem=recv_sem,
                device_id=(my - 1,),
                device_id_type=pl.DeviceIdType.MESH,
            )
            recv.wait_recv()
            offset_ref[...] = recv_buf[...]

        @pl.when(my < N_DEV - 1)
        def _():
            send_buf[...] = offset_ref[...] + total
            send = pltpu.make_async_remote_copy(
                src_ref=send_buf,
                dst_ref=recv_buf,
                send_sem=send_sem,
                recv_sem=recv_sem,
                device_id=(my + 1,),
                device_id_type=pl.DeviceIdType.MESH,
            )
            send.start()
            send.wait_send()

        off = offset_ref[...].astype(jnp.bfloat16)

        for i in range(nc):
            if i >= 2:
                out_copy(i - 2).wait()
            sl = pl.ds(i * CHUNK, CHUNK)
            res[sl, :] = res[sl, :] + off
            out_copy(i).start()
        out_copy(nc - 2).wait()
        out_copy(nc - 1).wait()

    return pl.pallas_call(
        body,
        out_shape=jax.ShapeDtypeStruct((m, n), jnp.bfloat16),
        in_specs=[pl.BlockSpec(memory_space=pl.ANY)],
        out_specs=pl.BlockSpec(memory_space=pl.ANY),
        scratch_shapes=[
            pltpu.VMEM((2, CHUNK, n), jnp.float32),
            pltpu.VMEM((m, n), jnp.bfloat16),
            pltpu.VMEM((1, n), jnp.float32),
            pltpu.VMEM((1, n), jnp.float32),
            pltpu.VMEM((1, n), jnp.float32),
            pltpu.SemaphoreType.DMA((2,)),
            pltpu.SemaphoreType.DMA((2,)),
            pltpu.SemaphoreType.DMA,
            pltpu.SemaphoreType.DMA,
        ],
        compiler_params=pltpu.CompilerParams(vmem_limit_bytes=40 * 1024 * 1024),
    )(x)


# device time: 42645 ns/iter; 1.4351x vs baseline; 1.4351x over previous
import jax
import jax.numpy as jnp
from jax import lax
from jax.experimental import pallas as pl
from jax.experimental.pallas import tpu as pltpu

N_DEV = 4
CHUNK = 256
NSEM = 4


def kernel(x):
    m, n = x.shape
    nc = m // CHUNK

    def body(x_ref, out_hbm, xb, send_buf, recv_bufs,
             out_sems, send_sems, recv_sems):
        my = lax.axis_index("i")

        row = lax.broadcasted_iota(jnp.int32, (CHUNK, CHUNK), 0)
        col = lax.broadcasted_iota(jnp.int32, (CHUNK, CHUNK), 1)
        tri = (row >= col).astype(jnp.bfloat16)

        o_local = []
        run = jnp.zeros((1, n), jnp.float32)
        for i in range(nc):
            o_local.append(run)
            run = run + jnp.sum(
                x_ref[pl.ds(i * CHUNK, CHUNK), :], axis=0, keepdims=True
            )
        total = run

        send_buf[...] = total

        def scatter_desc(src_slot, dst, dst_slot, sem):
            return pltpu.make_async_remote_copy(
                src_ref=send_buf,
                dst_ref=recv_bufs.at[dst_slot],
                send_sem=send_sems.at[sem],
                recv_sem=recv_sems.at[dst_slot],
                device_id=(dst,),
                device_id_type=pl.DeviceIdType.MESH,
            )

        for t in range(1, N_DEV):
            @pl.when(my + t <= N_DEV - 1)
            def _(t=t):
                scatter_desc(0, my + t, my, t - 1).start()

        for i in range(nc):
            sl = pl.ds(i * CHUNK, CHUNK)
            xb[sl, :] = x_ref[sl, :].astype(jnp.bfloat16)

        for j in range(N_DEV - 1):
            @pl.when(j < my)
            def _(j=j):
                scatter_desc(0, 0, j, 0).wait_recv()

        offset = jnp.zeros((1, n), jnp.float32)
        for j in range(N_DEV - 1):
            offset = offset + jnp.where(j < my, recv_bufs[j], 0.0)

        def out_copy(i):
            return pltpu.make_async_copy(
                xb.at[pl.ds(i * CHUNK, CHUNK), :],
                out_hbm.at[pl.ds(i * CHUNK, CHUNK), :],
                out_sems.at[i % NSEM],
            )

        for i in range(nc):
            if i >= NSEM:
                out_copy(i - NSEM).wait()
            sl = pl.ds(i * CHUNK, CHUNK)
            cum = lax.dot_general(
                tri, xb[sl, :],
                (((1,), (0,)), ((), ())),
                preferred_element_type=jnp.float32,
            )
            cum = cum + (o_local[i] + offset)
            xb[sl, :] = cum.astype(jnp.bfloat16)
            out_copy(i).start()
        for i in range(nc - NSEM, nc):
            out_copy(i).wait()

        for t in range(1, N_DEV):
            @pl.when(my + t <= N_DEV - 1)
            def _(t=t):
                scatter_desc(0, my + t, my, t - 1).wait_send()

    return pl.pallas_call(
        body,
        out_shape=jax.ShapeDtypeStruct((m, n), jnp.bfloat16),
        in_specs=[pl.BlockSpec(memory_space=pltpu.VMEM)],
        out_specs=pl.BlockSpec(memory_space=pl.ANY),
        scratch_shapes=[
            pltpu.VMEM((m, n), jnp.bfloat16),
            pltpu.VMEM((1, n), jnp.float32),
            pltpu.VMEM((N_DEV - 1, 1, n), jnp.float32),
            pltpu.SemaphoreType.DMA((NSEM,)),
            pltpu.SemaphoreType.DMA((N_DEV - 1,)),
            pltpu.SemaphoreType.DMA((N_DEV - 1,)),
        ],
        compiler_params=pltpu.CompilerParams(vmem_limit_bytes=56 * 1024 * 1024),
    )(x)


# device time: 41831 ns/iter; 1.4630x vs baseline; 1.0195x over previous
import jax
import jax.numpy as jnp
from jax import lax
from jax.experimental import pallas as pl
from jax.experimental.pallas import tpu as pltpu

N_DEV = 4
CHUNK = 256


def kernel(x):
    m, n = x.shape
    nc = m // CHUNK

    def body(x_ref, out_ref, send_buf, recv_bufs, send_sems, recv_sems):
        my = lax.axis_index("i")
        scope = jax.named_scope

        row = lax.broadcasted_iota(jnp.int32, (CHUNK, CHUNK), 0)
        col = lax.broadcasted_iota(jnp.int32, (CHUNK, CHUNK), 1)
        tri = (row >= col).astype(jnp.bfloat16)

        with scope("colsum"):
            o_local = []
            run = jnp.zeros((1, n), jnp.float32)
            for i in range(nc):
                o_local.append(run)
                run = run + jnp.sum(
                    x_ref[pl.ds(i * CHUNK, CHUNK), :], axis=0, keepdims=True
                )
            total = run

        send_buf[...] = total

        def scatter_desc(dst, dst_slot, sem):
            return pltpu.make_async_remote_copy(
                src_ref=send_buf,
                dst_ref=recv_bufs.at[dst_slot],
                send_sem=send_sems.at[sem],
                recv_sem=recv_sems.at[dst_slot],
                device_id=(dst,),
                device_id_type=pl.DeviceIdType.MESH,
            )

        with scope("scatter_send"):
            for t in range(1, N_DEV):
                @pl.when(my + t <= N_DEV - 1)
                def _(t=t):
                    scatter_desc(my + t, my, t - 1).start()

        with scope("wait_recv"):
            for j in range(N_DEV - 1):
                @pl.when(j < my)
                def _(j=j):
                    scatter_desc(0, j, 0).wait_recv()

            offset = jnp.zeros((1, n), jnp.float32)
            for j in range(N_DEV - 1):
                offset = offset + jnp.where(j < my, recv_bufs[j], 0.0)

        with scope("matmul_loop"):
            for i in range(nc):
                sl = pl.ds(i * CHUNK, CHUNK)
                cum = lax.dot_general(
                    tri, x_ref[sl, :].astype(jnp.bfloat16),
                    (((1,), (0,)), ((), ())),
                    preferred_element_type=jnp.float32,
                )
                cum = cum + (o_local[i] + offset)
                out_ref[sl, :] = cum.astype(jnp.bfloat16)

        for t in range(1, N_DEV):
            @pl.when(my + t <= N_DEV - 1)
            def _(t=t):
                scatter_desc(my + t, my, t - 1).wait_send()

    return pl.pallas_call(
        body,
        out_shape=jax.ShapeDtypeStruct((m, n), jnp.bfloat16),
        in_specs=[pl.BlockSpec(memory_space=pltpu.VMEM)],
        out_specs=pl.BlockSpec(memory_space=pltpu.VMEM),
        scratch_shapes=[
            pltpu.VMEM((1, n), jnp.float32),
            pltpu.VMEM((N_DEV - 1, 1, n), jnp.float32),
            pltpu.SemaphoreType.DMA((N_DEV - 1,)),
            pltpu.SemaphoreType.DMA((N_DEV - 1,)),
        ],
        compiler_params=pltpu.CompilerParams(vmem_limit_bytes=56 * 1024 * 1024),
    )(x)


# device time: 39394 ns/iter; 1.5535x vs baseline; 1.0619x over previous
import jax
import jax.numpy as jnp
from jax import lax
from jax.experimental import pallas as pl
from jax.experimental.pallas import tpu as pltpu

N_DEV = 4
CHUNK = 256
DCHUNK = 512
NBUF = 3
NOUT = 4


def kernel(x):
    m, n = x.shape
    nd = m // DCHUNK
    nc = m // CHUNK

    def body(x_hbm, out_hbm, xb, inbufs, send_buf, recv_bufs,
             in_sems, out_sems, send_sems, recv_sems):
        my = lax.axis_index("i")
        scope = jax.named_scope

        row = lax.broadcasted_iota(jnp.int32, (CHUNK, CHUNK), 0)
        col = lax.broadcasted_iota(jnp.int32, (CHUNK, CHUNK), 1)
        tri = (row >= col).astype(jnp.bfloat16)

        def in_copy(i):
            return pltpu.make_async_copy(
                x_hbm.at[pl.ds(i * DCHUNK, DCHUNK), :],
                inbufs.at[i % NBUF],
                in_sems.at[i % NBUF],
            )

        def out_copy(i):
            return pltpu.make_async_copy(
                xb.at[pl.ds(i * CHUNK, CHUNK), :],
                out_hbm.at[pl.ds(i * CHUNK, CHUNK), :],
                out_sems.at[i % NOUT],
            )

        with scope("phase1_stream_cumsum"):
            for i in range(min(NBUF, nd)):
                in_copy(i).start()
            carry = jnp.zeros((1, n), jnp.float32)
            for i in range(nd):
                in_copy(i).wait()
                for s in range(DCHUNK // CHUNK):
                    sub = inbufs[i % NBUF, pl.ds(s * CHUNK, CHUNK), :]
                    cum = lax.dot_general(
                        tri, sub.astype(jnp.bfloat16),
                        (((1,), (0,)), ((), ())),
                        preferred_element_type=jnp.float32,
                    )
                    cum = cum + carry
                    xb[pl.ds(i * DCHUNK + s * CHUNK, CHUNK), :] = (
                        cum.astype(jnp.bfloat16)
                    )
                    carry = cum[CHUNK - 1:CHUNK, :]
                if i + NBUF < nd:
                    in_copy(i + NBUF).start()
            total = carry

        send_buf[...] = total

        def scatter_desc(dst, dst_slot, sem):
            return pltpu.make_async_remote_copy(
                src_ref=send_buf,
                dst_ref=recv_bufs.at[dst_slot],
                send_sem=send_sems.at[sem],
                recv_sem=recv_sems.at[dst_slot],
                device_id=(dst,),
                device_id_type=pl.DeviceIdType.MESH,
            )

        with scope("scatter_send"):
            for t in range(1, N_DEV):
                @pl.when(my + t <= N_DEV - 1)
                def _(t=t):
                    scatter_desc(my + t, my, t - 1).start()

        with scope("wait_recv"):
            for j in range(N_DEV - 1):
                @pl.when(j < my)
                def _(j=j):
                    scatter_desc(0, j, 0).wait_recv()

            offset = jnp.zeros((1, n), jnp.float32)
            for j in range(N_DEV - 1):
                offset = offset + jnp.where(j < my, recv_bufs[j], 0.0)
            off_bf = offset.astype(jnp.bfloat16)

        with scope("phase3_offset_out"):
            for i in range(nc):
                if i >= NOUT:
                    out_copy(i - NOUT).wait()
                sl = pl.ds(i * CHUNK, CHUNK)
                xb[sl, :] = xb[sl, :] + off_bf
                out_copy(i).start()
        with scope("drain"):
            for i in range(max(0, nc - NOUT), nc):
                out_copy(i).wait()

        for t in range(1, N_DEV):
            @pl.when(my + t <= N_DEV - 1)
            def _(t=t):
                scatter_desc(my + t, my, t - 1).wait_send()

    return pl.pallas_call(
        body,
        out_shape=jax.ShapeDtypeStruct((m, n), jnp.bfloat16),
        in_specs=[pl.BlockSpec(memory_space=pl.ANY)],
        out_specs=pl.BlockSpec(memory_space=pl.ANY),
        scratch_shapes=[
            pltpu.VMEM((m, n), jnp.bfloat16),
            pltpu.VMEM((NBUF, DCHUNK, n), jnp.float32),
            pltpu.VMEM((1, n), jnp.float32),
            pltpu.VMEM((N_DEV - 1, 1, n), jnp.float32),
            pltpu.SemaphoreType.DMA((NBUF,)),
            pltpu.SemaphoreType.DMA((NOUT,)),
            pltpu.SemaphoreType.DMA((N_DEV - 1,)),
            pltpu.SemaphoreType.DMA((N_DEV - 1,)),
        ],
        compiler_params=pltpu.CompilerParams(vmem_limit_bytes=48 * 1024 * 1024),
    )(x)


# device time: 37027 ns/iter; 1.6528x vs baseline; 1.0639x over previous
import jax
import jax.numpy as jnp
from jax import lax
from jax.experimental import pallas as pl
from jax.experimental.pallas import tpu as pltpu

N_DEV = 4
CHUNK = 256
DCHUNK = 1024
NBUF = 3
OCHUNK = 512
NOUT = 6


def kernel(x):
    m, n = x.shape
    nd = m // DCHUNK
    nc = m // CHUNK

    def body(x_hbm, out_hbm, xb, inbufs, send_buf, recv_bufs,
             in_sems, out_sems, send_sems, recv_sems):
        my = lax.axis_index("i")
        scope = jax.named_scope

        row = lax.broadcasted_iota(jnp.int32, (CHUNK, CHUNK), 0)
        col = lax.broadcasted_iota(jnp.int32, (CHUNK, CHUNK), 1)
        tri = (row >= col).astype(jnp.bfloat16)

        def in_copy(i):
            return pltpu.make_async_copy(
                x_hbm.at[pl.ds(i * DCHUNK, DCHUNK), :],
                inbufs.at[i % NBUF],
                in_sems.at[i % NBUF],
            )

        def out_copy(i):
            return pltpu.make_async_copy(
                xb.at[pl.ds(i * OCHUNK, OCHUNK), :],
                out_hbm.at[pl.ds(i * OCHUNK, OCHUNK), :],
                out_sems.at[i % NOUT],
            )

        with scope("phase1_stream_cumsum"):
            for i in range(min(NBUF, nd)):
                in_copy(i).start()
            carry = jnp.zeros((1, n), jnp.float32)
            for i in range(nd):
                in_copy(i).wait()
                for s in range(DCHUNK // CHUNK):
                    sub = inbufs[i % NBUF, pl.ds(s * CHUNK, CHUNK), :]
                    cum = lax.dot_general(
                        tri, sub.astype(jnp.bfloat16),
                        (((1,), (0,)), ((), ())),
                        preferred_element_type=jnp.float32,
                    )
                    cum = cum + carry
                    xb[pl.ds(i * DCHUNK + s * CHUNK, CHUNK), :] = (
                        cum.astype(jnp.bfloat16)
                    )
                    carry = cum[CHUNK - 1:CHUNK, :]
                if i + NBUF < nd:
                    in_copy(i + NBUF).start()
            total = carry

        send_buf[...] = total

        def scatter_desc(dst, dst_slot, sem):
            return pltpu.make_async_remote_copy(
                src_ref=send_buf,
                dst_ref=recv_bufs.at[dst_slot],
                send_sem=send_sems.at[sem],
                recv_sem=recv_sems.at[dst_slot],
                device_id=(dst,),
                device_id_type=pl.DeviceIdType.MESH,
            )

        with scope("scatter_send"):
            for t in range(1, N_DEV):
                @pl.when(my + t <= N_DEV - 1)
                def _(t=t):
                    scatter_desc(my + t, my, t - 1).start()

        with scope("wait_recv"):
            for j in range(N_DEV - 1):
                @pl.when(j < my)
                def _(j=j):
                    scatter_desc(0, j, 0).wait_recv()

            offset = jnp.zeros((1, n), jnp.float32)
            for j in range(N_DEV - 1):
                offset = offset + jnp.where(j < my, recv_bufs[j], 0.0)
            off_bf = offset.astype(jnp.bfloat16)

        no = m // OCHUNK
        with scope("phase3_offset_out"):
            for i in range(no):
                if i >= NOUT:
                    out_copy(i - NOUT).wait()
                sl = pl.ds(i * OCHUNK, OCHUNK)
                xb[sl, :] = xb[sl, :] + off_bf
                out_copy(i).start()
        with scope("drain"):
            for i in range(max(0, no - NOUT), no):
                out_copy(i).wait()

        for t in range(1, N_DEV):
            @pl.when(my + t <= N_DEV - 1)
            def _(t=t):
                scatter_desc(my + t, my, t - 1).wait_send()

    return pl.pallas_call(
        body,
        out_shape=jax.ShapeDtypeStruct((m, n), jnp.bfloat16),
        in_specs=[pl.BlockSpec(memory_space=pl.ANY)],
        out_specs=pl.BlockSpec(memory_space=pl.ANY),
        scratch_shapes=[
            pltpu.VMEM((m, n), jnp.bfloat16),
            pltpu.VMEM((NBUF, DCHUNK, n), jnp.float32),
            pltpu.VMEM((1, n), jnp.float32),
            pltpu.VMEM((N_DEV - 1, 1, n), jnp.float32),
            pltpu.SemaphoreType.DMA((NBUF,)),
            pltpu.SemaphoreType.DMA((NOUT,)),
            pltpu.SemaphoreType.DMA((N_DEV - 1,)),
            pltpu.SemaphoreType.DMA((N_DEV - 1,)),
        ],
        compiler_params=pltpu.CompilerParams(vmem_limit_bytes=48 * 1024 * 1024),
    )(x)


# device time: 36467 ns/iter; 1.6782x vs baseline; 1.0154x over previous
import jax
import jax.numpy as jnp
from jax import lax
from jax.experimental import pallas as pl
from jax.experimental.pallas import tpu as pltpu

N_DEV = 4
CHUNK = 256
DCHUNK = 1024
NBUF = 3
NOUT = 6


def kernel(x):
    m, n = x.shape
    nd = m // DCHUNK
    nc = m // CHUNK

    def body(x_hbm, unoff_hbm, off_ref, stage, inbufs, send_buf, recv_bufs,
             in_sems, out_sems, send_sems, recv_sems):
        my = lax.axis_index("i")
        scope = jax.named_scope

        row = lax.broadcasted_iota(jnp.int32, (CHUNK, CHUNK), 0)
        col = lax.broadcasted_iota(jnp.int32, (CHUNK, CHUNK), 1)
        tri = (row >= col).astype(jnp.bfloat16)

        def in_copy(i):
            return pltpu.make_async_copy(
                x_hbm.at[pl.ds(i * DCHUNK, DCHUNK), :],
                inbufs.at[i % NBUF],
                in_sems.at[i % NBUF],
            )

        def out_copy(k):
            return pltpu.make_async_copy(
                stage.at[k % NOUT],
                unoff_hbm.at[pl.ds(k * CHUNK, CHUNK), :],
                out_sems.at[k % NOUT],
            )

        with scope("phase1_stream_cumsum"):
            for i in range(min(NBUF, nd)):
                in_copy(i).start()
            carry = jnp.zeros((1, n), jnp.float32)
            for i in range(nd):
                in_copy(i).wait()
                for s in range(DCHUNK // CHUNK):
                    k = i * (DCHUNK // CHUNK) + s
                    if k >= NOUT:
                        out_copy(k - NOUT).wait()
                    sub = inbufs[i % NBUF, pl.ds(s * CHUNK, CHUNK), :]
                    cum = lax.dot_general(
                        tri, sub.astype(jnp.bfloat16),
                        (((1,), (0,)), ((), ())),
                        preferred_element_type=jnp.float32,
                    )
                    cum = cum + carry
                    stage[k % NOUT] = cum.astype(jnp.bfloat16)
                    out_copy(k).start()
                    carry = cum[CHUNK - 1:CHUNK, :]
                if i + NBUF < nd:
                    in_copy(i + NBUF).start()
            total = carry

        send_buf[...] = total

        def scatter_desc(dst, dst_slot, sem):
            return pltpu.make_async_remote_copy(
                src_ref=send_buf,
                dst_ref=recv_bufs.at[dst_slot],
                send_sem=send_sems.at[sem],
                recv_sem=recv_sems.at[dst_slot],
                device_id=(dst,),
                device_id_type=pl.DeviceIdType.MESH,
            )

        with scope("scatter_send"):
            for t in range(1, N_DEV):
                @pl.when(my + t <= N_DEV - 1)
                def _(t=t):
                    scatter_desc(my + t, my, t - 1).start()

        with scope("wait_recv"):
            for j in range(N_DEV - 1):
                @pl.when(j < my)
                def _(j=j):
                    scatter_desc(0, j, 0).wait_recv()

            offset = jnp.zeros((1, n), jnp.float32)
            for j in range(N_DEV - 1):
                offset = offset + jnp.where(j < my, recv_bufs[j], 0.0)
            off_ref[...] = offset

        with scope("drain"):
            for k in range(max(0, nc - NOUT), nc):
                out_copy(k).wait()
            for t in range(1, N_DEV):
                @pl.when(my + t <= N_DEV - 1)
                def _(t=t):
                    scatter_desc(my + t, my, t - 1).wait_send()

    unoff, off = pl.pallas_call(
        body,
        out_shape=[
            jax.ShapeDtypeStruct((m, n), jnp.bfloat16),
            jax.ShapeDtypeStruct((1, n), jnp.float32),
        ],
        in_specs=[pl.BlockSpec(memory_space=pl.ANY)],
        out_specs=[
            pl.BlockSpec(memory_space=pl.ANY),
            pl.BlockSpec(memory_space=pltpu.VMEM),
        ],
        scratch_shapes=[
            pltpu.VMEM((NOUT, CHUNK, n), jnp.bfloat16),
            pltpu.VMEM((NBUF, DCHUNK, n), jnp.float32),
            pltpu.VMEM((1, n), jnp.float32),
            pltpu.VMEM((N_DEV - 1, 1, n), jnp.float32),
            pltpu.SemaphoreType.DMA((NBUF,)),
            pltpu.SemaphoreType.DMA((NOUT,)),
            pltpu.SemaphoreType.DMA((N_DEV - 1,)),
            pltpu.SemaphoreType.DMA((N_DEV - 1,)),
        ],
        compiler_params=pltpu.CompilerParams(vmem_limit_bytes=40 * 1024 * 1024),
    )(x)

    return unoff + off.astype(jnp.bfloat16)


# device time: 36274 ns/iter; 1.6871x vs baseline; 1.0053x over previous
import jax
import jax.numpy as jnp
from jax import lax
from jax.experimental import pallas as pl
from jax.experimental.pallas import tpu as pltpu

N_DEV = 4
CHUNK = 256
DCHUNK = 1024
NBUF = 4
NOUT = 8


def kernel(x):
    m, n = x.shape
    nd = m // DCHUNK
    nc = m // CHUNK

    def body(x_hbm, unoff_hbm, off_ref, stage, inbufs, send_buf, recv_bufs,
             in_sems, out_sems, send_sems, recv_sems):
        my = lax.axis_index("i")
        scope = jax.named_scope

        row = lax.broadcasted_iota(jnp.int32, (CHUNK, CHUNK), 0)
        col = lax.broadcasted_iota(jnp.int32, (CHUNK, CHUNK), 1)
        tri = (row >= col).astype(jnp.bfloat16)

        def in_copy(i):
            return pltpu.make_async_copy(
                x_hbm.at[pl.ds(i * DCHUNK, DCHUNK), :],
                inbufs.at[i % NBUF],
                in_sems.at[i % NBUF],
            )

        def out_copy(k):
            return pltpu.make_async_copy(
                stage.at[k % NOUT],
                unoff_hbm.at[pl.ds(k * CHUNK, CHUNK), :],
                out_sems.at[k % NOUT],
            )

        with scope("phase1_stream_cumsum"):
            for i in range(min(NBUF, nd)):
                in_copy(i).start()
            carry = jnp.zeros((1, n), jnp.float32)
            for i in range(nd):
                in_copy(i).wait()
                for s in range(DCHUNK // CHUNK):
                    k = i * (DCHUNK // CHUNK) + s
                    if k >= NOUT:
                        out_copy(k - NOUT).wait()
                    sub = inbufs[i % NBUF, pl.ds(s * CHUNK, CHUNK), :]
                    cum = lax.dot_general(
                        tri, sub.astype(jnp.bfloat16),
                        (((1,), (0,)), ((), ())),
                        preferred_element_type=jnp.float32,
                    )
                    cum = cum + carry
                    stage[k % NOUT] = cum.astype(jnp.bfloat16)
                    out_copy(k).start()
                    carry = cum[CHUNK - 1:CHUNK, :]
                if i + NBUF < nd:
                    in_copy(i + NBUF).start()
            total = carry

        send_buf[...] = total

        def scatter_desc(dst, dst_slot, sem):
            return pltpu.make_async_remote_copy(
                src_ref=send_buf,
                dst_ref=recv_bufs.at[dst_slot],
                send_sem=send_sems.at[sem],
                recv_sem=recv_sems.at[dst_slot],
                device_id=(dst,),
                device_id_type=pl.DeviceIdType.MESH,
            )

        with scope("scatter_send"):
            for t in range(1, N_DEV):
                @pl.when(my + t <= N_DEV - 1)
                def _(t=t):
                    scatter_desc(my + t, my, t - 1).start()

        with scope("wait_recv"):
            for j in range(N_DEV - 1):
                @pl.when(j < my)
                def _(j=j):
                    scatter_desc(0, j, 0).wait_recv()

            offset = jnp.zeros((1, n), jnp.float32)
            for j in range(N_DEV - 1):
                offset = offset + jnp.where(j < my, recv_bufs[j], 0.0)
            off_ref[...] = offset

        with scope("drain"):
            for k in range(max(0, nc - NOUT), nc):
                out_copy(k).wait()
            for t in range(1, N_DEV):
                @pl.when(my + t <= N_DEV - 1)
                def _(t=t):
                    scatter_desc(my + t, my, t - 1).wait_send()

    unoff, off = pl.pallas_call(
        body,
        out_shape=[
            jax.ShapeDtypeStruct((m, n), jnp.bfloat16),
            jax.ShapeDtypeStruct((1, n), jnp.float32),
        ],
        in_specs=[pl.BlockSpec(memory_space=pl.ANY)],
        out_specs=[
            pl.BlockSpec(memory_space=pl.ANY),
            pl.BlockSpec(memory_space=pltpu.VMEM),
        ],
        scratch_shapes=[
            pltpu.VMEM((NOUT, CHUNK, n), jnp.bfloat16),
            pltpu.VMEM((NBUF, DCHUNK, n), jnp.float32),
            pltpu.VMEM((1, n), jnp.float32),
            pltpu.VMEM((N_DEV - 1, 1, n), jnp.float32),
            pltpu.SemaphoreType.DMA((NBUF,)),
            pltpu.SemaphoreType.DMA((NOUT,)),
            pltpu.SemaphoreType.DMA((N_DEV - 1,)),
            pltpu.SemaphoreType.DMA((N_DEV - 1,)),
        ],
        compiler_params=pltpu.CompilerParams(vmem_limit_bytes=40 * 1024 * 1024),
    )(x)

    return unoff + off.astype(jnp.bfloat16)
